# Initial kernel scaffold; baseline (speedup 1.0000x reference)
import jax
import jax.numpy as jnp
from jax.experimental import pallas as pl


def kernel(features):
    raise NotImplementedError("write your pallas kernel here")


# speedup vs baseline: 33.5437x; 33.5437x over previous
import functools

import jax
import jax.numpy as jnp
from jax.experimental import pallas as pl

_K = 256
_INT_MIN = -(2**31)


def _topk_mask_kernel(x_ref, o_ref):
    x = x_ref[...]
    b = jax.lax.bitcast_convert_type(x, jnp.int32)
    keys = jnp.where(b < 0, b ^ jnp.int32(0x7FFFFFFF), b)

    rows = x.shape[0]

    def body(j, u):
        bit = jnp.left_shift(jnp.int32(1), jnp.int32(31) - j)
        cand_u = u | bit
        cand_s = cand_u ^ jnp.int32(_INT_MIN)
        cnt = jnp.sum((keys >= cand_s).astype(jnp.int32), axis=1, keepdims=True)
        return jnp.where(cnt >= _K, cand_u, u)

    u0 = jnp.zeros((rows, 1), jnp.int32)
    u_star = jax.lax.fori_loop(0, 32, body, u0)
    thr = u_star ^ jnp.int32(_INT_MIN)
    o_ref[...] = jnp.where(keys >= thr, x, jnp.float32(0.0))


@jax.jit
def kernel(features):
    B, L, D = features.shape
    x = features.reshape(B * L, D)
    rows_per_block = 128
    while x.shape[0] % rows_per_block:
        rows_per_block //= 2
    grid = (x.shape[0] // rows_per_block,)
    out = pl.pallas_call(
        _topk_mask_kernel,
        out_shape=jax.ShapeDtypeStruct(x.shape, x.dtype),
        grid=grid,
        in_specs=[pl.BlockSpec((rows_per_block, D), lambda i: (i, 0))],
        out_specs=pl.BlockSpec((rows_per_block, D), lambda i: (i, 0)),
    )(x)
    return out.reshape(B, L, D)
